# Initial kernel scaffold; baseline (speedup 1.0000x reference)
import jax
import jax.numpy as jnp
from jax.experimental import pallas as pl


def kernel(token_ids, segment_ids, table):
    raise NotImplementedError("write your pallas kernel here")


# speedup vs baseline: 1.2011x; 1.2011x over previous
import jax
import jax.numpy as jnp
from jax import lax
from jax.experimental import pallas as pl
from jax.experimental.pallas import tpu as pltpu
from jax.experimental.pallas import tpu_sc as plsc

_VOCAB = 100000
_DIM = 300
_N_TOK = 65536
_N_SENT = 4096
_N_TEXT = 16
_SPT = _N_SENT // _N_TEXT

_NC = 2
_NS = 16
_NW = _NC * _NS
_L = 16

_DPAD = 304
_NSL = _DPAD // _L
_GCH = 64
_TOK_PER_W = _N_TOK // _NW
_N_CH = _TOK_PER_W // _GCH
_CNT_PER_S = _N_TOK // _NS
_N_CNT_CH = _CNT_PER_S // _GCH
_CNT_W = 16
_SENT_PER_S = _N_SENT // _NS


def _sc_body(tok2d, seg2d, table, ones_hbm, zrow_hbm, zcnt_hbm, part_out,
             tok_idx, seg_idx, cseg_idx, rows, ones_v, cnt_v, outbuf,
             sent_sum, sent_cnt, sem):
    c = lax.axis_index("c")
    s = lax.axis_index("s")
    wid = c * _NS + s

    pltpu.sync_copy(tok2d.at[pl.ds(wid * _N_CH, _N_CH)], tok_idx)
    pltpu.sync_copy(seg2d.at[pl.ds(wid * _N_CH, _N_CH)], seg_idx)
    pltpu.sync_copy(seg2d.at[pl.ds(s * _N_CNT_CH, _N_CNT_CH)], cseg_idx)
    pltpu.sync_copy(ones_hbm, ones_v)

    pltpu.sync_copy(zrow_hbm, sent_sum.at[pl.ds(s * _SENT_PER_S, _SENT_PER_S)])
    pltpu.sync_copy(zcnt_hbm, sent_cnt.at[pl.ds(s * _SENT_PER_S, _SENT_PER_S)])
    plsc.subcore_barrier()

    def cnt_body(j, carry):
        pltpu.sync_copy(ones_v, sent_cnt.at[cseg_idx.at[j]], add=True)
        return carry
    lax.fori_loop(0, _N_CNT_CH, cnt_body, 0)

    def gs_body(j, carry):
        pltpu.async_copy(table.at[tok_idx.at[j]], rows, sem).wait()
        pltpu.sync_copy(rows, sent_sum.at[seg_idx.at[j]], add=True)
        return carry
    lax.fori_loop(0, _N_CH, gs_body, 0)
    plsc.subcore_barrier()

    pltpu.sync_copy(sent_cnt.at[pl.ds(s * _SENT_PER_S, _SENT_PER_S)], cnt_v)
    acc = tuple(jnp.zeros((_L,), jnp.float32) for _ in range(_NSL))
    for half in range(_SENT_PER_S // _GCH):
        pltpu.sync_copy(
            sent_sum.at[pl.ds(s * _SENT_PER_S + half * _GCH, _GCH)], rows)

        def row_body(r, a, half=half):
            cvec = cnt_v[half * _GCH + r, :]
            scale = 1.0 / (float(_SPT) * jnp.maximum(cvec, 1.0))
            return tuple(a[j] + scale * rows[r, pl.ds(j * _L, _L)]
                         for j in range(_NSL))
        acc = lax.fori_loop(0, _GCH, row_body, acc)

    for j in range(_NSL):
        outbuf[pl.ds(j * _L, _L)] = acc[j]
    pltpu.sync_copy(outbuf, part_out.at[c, s])


@jax.jit
def _sc_encode(tok2d, seg2d, table, ones, zrow, zcnt):
    mesh = plsc.VectorSubcoreMesh(core_axis_name="c", subcore_axis_name="s")
    return pl.kernel(
        _sc_body,
        out_type=jax.ShapeDtypeStruct((_NC, _N_TEXT, _DPAD), jnp.float32),
        mesh=mesh,
        compiler_params=pltpu.CompilerParams(use_tc_tiling_on_sc=False),
        scratch_types=[
            pltpu.VMEM((_N_CH, _GCH), jnp.int32),
            pltpu.VMEM((_N_CH, _GCH), jnp.int32),
            pltpu.VMEM((_N_CNT_CH, _GCH), jnp.int32),
            pltpu.VMEM((_GCH, _DPAD), jnp.float32),
            pltpu.VMEM((_GCH, _CNT_W), jnp.float32),
            pltpu.VMEM((_SENT_PER_S, _CNT_W), jnp.float32),
            pltpu.VMEM((_DPAD,), jnp.float32),
            pltpu.VMEM_SHARED((_N_SENT, _DPAD), jnp.float32),
            pltpu.VMEM_SHARED((_N_SENT, _CNT_W), jnp.float32),
            pltpu.SemaphoreType.DMA,
        ],
    )(tok2d, seg2d, table, ones, zrow, zcnt)


def _combine_body(p_ref, o_ref):
    o_ref[...] = p_ref[0, :, :_DIM] + p_ref[1, :, :_DIM]


@jax.jit
def _combine(part):
    return pl.pallas_call(
        _combine_body,
        out_shape=jax.ShapeDtypeStruct((_N_TEXT, _DIM), jnp.float32),
    )(part)


def kernel(token_ids, segment_ids, table):
    table_p = jnp.pad(table, ((0, 0), (0, _DPAD - _DIM)))
    tok2d = token_ids.reshape(_N_TOK // _GCH, _GCH)
    seg2d = segment_ids.reshape(_N_TOK // _GCH, _GCH)
    ones = jnp.ones((_GCH, _CNT_W), jnp.float32)
    zrow = jnp.zeros((_SENT_PER_S, _DPAD), jnp.float32)
    zcnt = jnp.zeros((_SENT_PER_S, _CNT_W), jnp.float32)
    part = _sc_encode(tok2d, seg2d, table_p, ones, zrow, zcnt)
    return _combine(part)
